# baseline (speedup 1.0000x reference)
import jax
import jax.numpy as jnp
from jax import lax
from jax.experimental import pallas as pl
from jax.experimental.pallas import tpu as pltpu
from jax.experimental.pallas import tpu_sc as plsc

N_NODES = 10000
N_PAD = 10240
D = 128
E = 320000
NC, NS = 2, 16
NW = NC * NS
CHUNK = 128
CH_TOT = E // CHUNK
CH_PER_W = 79
ROWS_PER_TILE = N_PAD // NS


def _seg_sum_body(x_hbm, ei_hbm, out_hbm,
                  srci, dsti, rows0, rows1, acc,
                  sis, dis, gsem, ssem, zsem):
    c = lax.axis_index("c")
    s = lax.axis_index("s")
    w = c * NS + s
    rows = [rows0, rows1]

    def chunk_base(j):
        cc = w + NW * j
        return pl.multiple_of(
            jnp.where(cc < CH_TOT, cc, w) * CHUNK, 8), cc

    zeros16 = jnp.zeros((16,), jnp.float32)

    def zero_rows(buf):
        def zbody(i, _):
            r = i // (D // 16)
            col = (i % (D // 16)) * 16
            buf[r, pl.ds(col, 16)] = zeros16
            return 0
        lax.fori_loop(0, CHUNK * (D // 16), zbody, 0)

    zero_rows(rows0)
    zd = [pltpu.async_copy(
        rows0, acc.at[pl.ds(s * ROWS_PER_TILE + t * CHUNK, CHUNK)], zsem)
        for t in range(ROWS_PER_TILE // CHUNK)]
    for d in zd:
        d.wait()
    plsc.subcore_barrier()

    def idx_issue(slot, j):
        b, _ = chunk_base(j)
        pltpu.async_copy(ei_hbm.at[0, pl.ds(b, CHUNK)], srci.at[slot],
                         sis.at[slot])
        pltpu.async_copy(ei_hbm.at[1, pl.ds(b, CHUNK)], dsti.at[slot],
                         dis.at[slot])

    def idx_wait_src(slot, j):
        b, _ = chunk_base(j)
        pltpu.make_async_copy(ei_hbm.at[0, pl.ds(b, CHUNK)], srci.at[slot],
                              sis.at[slot]).wait()

    def idx_wait_dst(slot, j):
        b, _ = chunk_base(j)
        pltpu.make_async_copy(ei_hbm.at[1, pl.ds(b, CHUNK)], dsti.at[slot],
                              dis.at[slot]).wait()

    def step(j, t, drain, prefetch, gather):
        bank = t % 2
        if drain is not False:
            def _drain():
                pltpu.make_async_copy(rows[1 - bank],
                                      acc.at[dsti.at[(t + 3) % 4]],
                                      ssem.at[1 - bank]).wait()
            if drain is True:
                _drain()
            else:
                pl.when(drain)(_drain)
        if prefetch:
            idx_issue((t + 2) % 4, j + 2)
        if gather:
            idx_wait_src((t + 1) % 4, j + 1)
            pltpu.async_copy(x_hbm.at[srci.at[(t + 1) % 4]],
                             rows[1 - bank], gsem.at[1 - bank])
        pltpu.make_async_copy(x_hbm.at[srci.at[t]], rows[bank],
                              gsem.at[bank]).wait()
        idx_wait_dst(t, j)
        _, cc = chunk_base(j)

        @pl.when(cc >= CH_TOT)
        def _():
            zero_rows(rows[bank])

        pltpu.async_copy(rows[bank], acc.at[dsti.at[t]],
                         ssem.at[bank], add=True)

    idx_issue(0, 0)
    idx_issue(1, 1)
    idx_wait_src(0, 0)
    pltpu.async_copy(x_hbm.at[srci.at[0]], rows[0], gsem.at[0])

    def body(i4, _):
        j0 = 4 * i4
        step(j0, 0, i4 > 0, True, True)
        step(j0 + 1, 1, True, True, True)
        step(j0 + 2, 2, True, True, True)
        step(j0 + 3, 3, True, True, True)
        return 0

    lax.fori_loop(0, CH_PER_W // 4, body, 0)
    step(76, 0, True, True, True)
    step(77, 1, True, False, True)
    step(78, 2, True, False, False)
    pltpu.make_async_copy(rows[0], acc.at[dsti.at[2]], ssem.at[0]).wait()
    plsc.subcore_barrier()

    r0 = s * ROWS_PER_TILE
    pltpu.sync_copy(acc.at[pl.ds(r0, ROWS_PER_TILE)],
                    out_hbm.at[c, pl.ds(r0, ROWS_PER_TILE)])


_seg_sum = pl.kernel(
    _seg_sum_body,
    out_type=jax.ShapeDtypeStruct((NC, N_PAD, D), jnp.float32),
    mesh=plsc.VectorSubcoreMesh(core_axis_name="c", subcore_axis_name="s"),
    scratch_types=[
        pltpu.VMEM((4, CHUNK), jnp.int32),
        pltpu.VMEM((4, CHUNK), jnp.int32),
        pltpu.VMEM((CHUNK, D), jnp.float32),
        pltpu.VMEM((CHUNK, D), jnp.float32),
        pltpu.VMEM_SHARED((N_PAD, D), jnp.float32),
        pltpu.SemaphoreType.DMA((4,)),
        pltpu.SemaphoreType.DMA((4,)),
        pltpu.SemaphoreType.DMA((2,)),
        pltpu.SemaphoreType.DMA((2,)),
        pltpu.SemaphoreType.DMA,
    ],
)


def _layer1_body(p_ref, W_ref, b_ref, o_ref):
    a = p_ref[0] + p_ref[1]
    o_ref[...] = jnp.maximum(
        jnp.dot(a, W_ref[...], preferred_element_type=jnp.float32)
        + b_ref[...], 0.0)


def _layer2_body(p_ref, W_ref, b_ref, hw_ref, hb_ref, o_ref):
    a = p_ref[0] + p_ref[1]
    h = jnp.maximum(
        jnp.dot(a, W_ref[...], preferred_element_type=jnp.float32)
        + b_ref[...], 0.0)
    lg = jnp.dot(h, hw_ref[...],
                 preferred_element_type=jnp.float32) + hb_ref[...]
    o_ref[...] = lg[:N_NODES, 0]


def _tc_layer1(p, W, b):
    return pl.pallas_call(
        _layer1_body,
        out_shape=jax.ShapeDtypeStruct((N_PAD, D), jnp.float32),
    )(p, W, b)


def _tc_layer2(p, W, b, head_w, head_b):
    return pl.pallas_call(
        _layer2_body,
        out_shape=jax.ShapeDtypeStruct((N_NODES,), jnp.float32),
    )(p, W, b, head_w, head_b)


def kernel(x, edge_index, W1, b1, W2, b2, head_w, head_b):
    agg1 = _seg_sum(x, edge_index)
    h1 = _tc_layer1(agg1, W1, b1.reshape(1, D))
    agg2 = _seg_sum(h1, edge_index)
    return _tc_layer2(agg2, W2, b2.reshape(1, D),
                      head_w, head_b.reshape(1, 1))


# speedup vs baseline: 1.0151x; 1.0151x over previous
import jax
import jax.numpy as jnp
from jax import lax
from jax.experimental import pallas as pl
from jax.experimental.pallas import tpu as pltpu
from jax.experimental.pallas import tpu_sc as plsc

N_NODES = 10000
N_PAD = 10240
D = 128
E = 320000
NC, NS = 2, 16
NW = NC * NS
CHUNK = 128
CH_TOT = E // CHUNK
CH_PER_W = 79
ROWS_PER_TILE = N_PAD // NS


def _seg_sum_body(x_hbm, ei_hbm, out_hbm,
                  srci, dsti, rows0, rows1, acc,
                  sis, dis, gsem, ssem, zsem):
    c = lax.axis_index("c")
    s = lax.axis_index("s")
    w = c * NS + s
    rows = [rows0, rows1]

    def chunk_base(j):
        cc = w + NW * j
        return pl.multiple_of(
            jnp.where(cc < CH_TOT, cc, w) * CHUNK, 8), cc

    zeros16 = jnp.zeros((16,), jnp.float32)

    def zero_rows(buf):
        def zbody(i, _):
            r = i // (D // 16)
            col = (i % (D // 16)) * 16
            buf[r, pl.ds(col, 16)] = zeros16
            return 0
        lax.fori_loop(0, CHUNK * (D // 16), zbody, 0)

    zero_rows(rows0)
    zd = [pltpu.async_copy(
        rows0, acc.at[pl.ds(s * ROWS_PER_TILE + t * CHUNK, CHUNK)], zsem)
        for t in range(ROWS_PER_TILE // CHUNK)]
    for d in zd:
        d.wait()
    plsc.subcore_barrier()

    def idx_issue(slot, j):
        b, _ = chunk_base(j)
        pltpu.async_copy(ei_hbm.at[0, pl.ds(b, CHUNK)], srci.at[slot],
                         sis.at[slot])
        pltpu.async_copy(ei_hbm.at[1, pl.ds(b, CHUNK)], dsti.at[slot],
                         dis.at[slot])

    def idx_wait_src(slot, j):
        b, _ = chunk_base(j)
        pltpu.make_async_copy(ei_hbm.at[0, pl.ds(b, CHUNK)], srci.at[slot],
                              sis.at[slot]).wait()

    def idx_wait_dst(slot, j):
        b, _ = chunk_base(j)
        pltpu.make_async_copy(ei_hbm.at[1, pl.ds(b, CHUNK)], dsti.at[slot],
                              dis.at[slot]).wait()

    def step(j, t, drain, prefetch, gather):
        bank = t % 2
        if drain is not False:
            def _drain():
                pltpu.make_async_copy(rows[1 - bank],
                                      acc.at[dsti.at[(t + 3) % 4]],
                                      ssem.at[1 - bank]).wait()
            if drain is True:
                _drain()
            else:
                pl.when(drain)(_drain)
        if prefetch:
            idx_issue((t + 2) % 4, j + 2)
        if gather:
            idx_wait_src((t + 1) % 4, j + 1)
            pltpu.async_copy(x_hbm.at[srci.at[(t + 1) % 4]],
                             rows[1 - bank], gsem.at[1 - bank])
        pltpu.make_async_copy(x_hbm.at[srci.at[t]], rows[bank],
                              gsem.at[bank]).wait()
        idx_wait_dst(t, j)
        _, cc = chunk_base(j)

        @pl.when(cc >= CH_TOT)
        def _():
            zero_rows(rows[bank])

        pltpu.async_copy(rows[bank], acc.at[dsti.at[t]],
                         ssem.at[bank], add=True)

    idx_issue(0, 0)
    idx_issue(1, 1)
    idx_wait_src(0, 0)
    pltpu.async_copy(x_hbm.at[srci.at[0]], rows[0], gsem.at[0])

    def body(i4, _):
        j0 = 4 * i4
        step(j0, 0, i4 > 0, True, True)
        step(j0 + 1, 1, True, True, True)
        step(j0 + 2, 2, True, True, True)
        step(j0 + 3, 3, True, True, True)
        return 0

    lax.fori_loop(0, CH_PER_W // 4, body, 0)
    step(76, 0, True, True, True)
    step(77, 1, True, False, True)
    step(78, 2, True, False, False)
    pltpu.make_async_copy(rows[0], acc.at[dsti.at[2]], ssem.at[0]).wait()
    plsc.subcore_barrier()

    wd = []
    for t in range(ROWS_PER_TILE // CHUNK):
        k = t % 2
        r0 = s * ROWS_PER_TILE + t * CHUNK
        if t >= 2:
            wd[t - 2].wait()
        pltpu.sync_copy(acc.at[pl.ds(r0, CHUNK)], rows[k])
        wd.append(pltpu.async_copy(rows[k], out_hbm.at[c, pl.ds(r0, CHUNK)],
                                   ssem.at[k]))
    for t in range(max(0, ROWS_PER_TILE // CHUNK - 2),
                   ROWS_PER_TILE // CHUNK):
        wd[t].wait()


_seg_sum = pl.kernel(
    _seg_sum_body,
    out_type=jax.ShapeDtypeStruct((NC, N_PAD, D), jnp.float32),
    mesh=plsc.VectorSubcoreMesh(core_axis_name="c", subcore_axis_name="s"),
    scratch_types=[
        pltpu.VMEM((4, CHUNK), jnp.int32),
        pltpu.VMEM((4, CHUNK), jnp.int32),
        pltpu.VMEM((CHUNK, D), jnp.float32),
        pltpu.VMEM((CHUNK, D), jnp.float32),
        pltpu.VMEM_SHARED((N_PAD, D), jnp.float32),
        pltpu.SemaphoreType.DMA((4,)),
        pltpu.SemaphoreType.DMA((4,)),
        pltpu.SemaphoreType.DMA((2,)),
        pltpu.SemaphoreType.DMA((2,)),
        pltpu.SemaphoreType.DMA,
    ],
)


def _layer1_body(p_ref, W_ref, b_ref, o_ref):
    a = p_ref[0] + p_ref[1]
    o_ref[...] = jnp.maximum(
        jnp.dot(a, W_ref[...], preferred_element_type=jnp.float32)
        + b_ref[...], 0.0)


def _layer2_body(p_ref, W_ref, b_ref, hw_ref, hb_ref, o_ref):
    a = p_ref[0] + p_ref[1]
    h = jnp.maximum(
        jnp.dot(a, W_ref[...], preferred_element_type=jnp.float32)
        + b_ref[...], 0.0)
    lg = jnp.dot(h, hw_ref[...],
                 preferred_element_type=jnp.float32) + hb_ref[...]
    o_ref[...] = lg[:N_NODES, 0]


_BN = 2560


def _tc_layer1(p, W, b):
    return pl.pallas_call(
        _layer1_body,
        grid=(N_PAD // _BN,),
        in_specs=[
            pl.BlockSpec((NC, _BN, D), lambda i: (0, i, 0)),
            pl.BlockSpec((D, D), lambda i: (0, 0)),
            pl.BlockSpec((1, D), lambda i: (0, 0)),
        ],
        out_specs=pl.BlockSpec((_BN, D), lambda i: (i, 0)),
        out_shape=jax.ShapeDtypeStruct((N_PAD, D), jnp.float32),
    )(p, W, b)


def _tc_layer2(p, W, b, head_w, head_b):
    return pl.pallas_call(
        _layer2_body,
        out_shape=jax.ShapeDtypeStruct((N_NODES,), jnp.float32),
    )(p, W, b, head_w, head_b)


def kernel(x, edge_index, W1, b1, W2, b2, head_w, head_b):
    agg1 = _seg_sum(x, edge_index)
    h1 = _tc_layer1(agg1, W1, b1.reshape(1, D))
    agg2 = _seg_sum(h1, edge_index)
    return _tc_layer2(agg2, W2, b2.reshape(1, D),
                      head_w, head_b.reshape(1, 1))
